# baseline (speedup 1.0000x reference)
import functools

import jax
import jax.numpy as jnp
from jax import lax
from jax.experimental import pallas as pl
from jax.experimental.pallas import tpu as pltpu
from jax.experimental.pallas import tpu_sc as plsc

RES = 128
RADIUS = 1.3
SH_DIM = 9
DATA_DIM = 28
VOXEL = RADIUS * 2 / RES
STEP = VOXEL / 2
N_INTRS = 443
BATCH = 1024
NPTS = BATCH * N_INTRS
C = 32
NVOX = RES * RES * RES

C0 = 0.28209479177387814
C1 = 0.4886025119029199
C2 = (1.0925484305920792, -1.0925484305920792, 0.31539156525252005,
      -1.0925484305920792, 0.5462742152960396)

NSC = 2
NSUB = 16
NW = NSC * NSUB
PPS = NPTS // NW
PB = 32
NIT = PPS // PB


def _ray_bounds(o, d):
    inv = 1.0 / d
    t1 = (-RADIUS - o) * inv
    t2 = (RADIUS - o) * inv
    tnear = jnp.maximum(jnp.max(jnp.minimum(t1, t2), axis=-1), 0.0)
    tfar = jnp.min(jnp.maximum(t1, t2), axis=-1)
    return tnear, tfar


def _prep_body(o_ref, d_ref, idx_ref, w_ref):
    o = o_ref[...]
    d = d_ref[...]
    rb = o.shape[0]
    tnear, tfar = _ray_bounds(o, d)
    k = lax.broadcasted_iota(jnp.int32, (rb, N_INTRS), 1).astype(jnp.float32)
    ints0 = tnear[:, None] + k * STEP
    ints1 = tnear[:, None] + (k + 1.0) * STEP
    tmid = 0.5 * (ints0 + ints1)
    mask = tmid < tfar[:, None]
    i0s, i1s, ws = [], [], []
    for a in range(3):
        pa = o[:, a:a + 1] + d[:, a:a + 1] * tmid
        mask = mask & (jnp.abs(pa) <= RADIUS)
        g = (pa / RADIUS + 1.0) * 0.5 * (RES - 1)
        g0 = jnp.floor(g)
        ws.append(g - g0)
        gi = g0.astype(jnp.int32)
        i0s.append(jnp.clip(gi, 0, RES - 1))
        i1s.append(jnp.clip(gi + 1, 0, RES - 1))
    mf = mask.astype(jnp.float32)
    wx, wy, wz = ws
    for j in range(8):
        jx, jy, jz = j & 1, (j >> 1) & 1, (j >> 2) & 1
        ix = i1s[0] if jx else i0s[0]
        iy = i1s[1] if jy else i0s[1]
        iz = i1s[2] if jz else i0s[2]
        wj = ((wx if jx else 1.0 - wx)
              * (wy if jy else 1.0 - wy)
              * (wz if jz else 1.0 - wz) * mf)
        idx_ref[j] = (iz * RES + iy) * RES + ix
        w_ref[j] = wj


def _prep(rays_o, rays_d):
    rb = 128
    grid = (BATCH // rb,)
    return pl.pallas_call(
        _prep_body,
        grid=grid,
        in_specs=[pl.BlockSpec((rb, 3), lambda r: (r, 0)),
                  pl.BlockSpec((rb, 3), lambda r: (r, 0))],
        out_specs=[pl.BlockSpec((8, rb, N_INTRS), lambda r: (0, r, 0)),
                   pl.BlockSpec((8, rb, N_INTRS), lambda r: (0, r, 0))],
        out_shape=[jax.ShapeDtypeStruct((8, BATCH, N_INTRS), jnp.int32),
                   jax.ShapeDtypeStruct((8, BATCH, N_INTRS), jnp.float32)],
    )(rays_o, rays_d)


BLK = 8 * PB


def _sc_interp(table, idx, w):
    mesh = plsc.VectorSubcoreMesh(core_axis_name="c", subcore_axis_name="s")

    @functools.partial(
        pl.kernel, mesh=mesh,
        compiler_params=pltpu.CompilerParams(use_tc_tiling_on_sc=False),
        out_type=jax.ShapeDtypeStruct((NPTS, C), jnp.float32),
        scratch_types=[
            pltpu.VMEM((2, BLK), jnp.int32),
            pltpu.VMEM((2, BLK), jnp.float32),
            pltpu.VMEM((2, 8, PB, C), jnp.float32),
            pltpu.VMEM((2, PB, C), jnp.float32),
            pltpu.SemaphoreType.DMA,
            pltpu.SemaphoreType.DMA,
            pltpu.SemaphoreType.DMA,
            pltpu.SemaphoreType.DMA,
            pltpu.SemaphoreType.DMA,
            pltpu.SemaphoreType.DMA,
        ],
    )
    def k(table_h, idx_h, w_h, out_h, idx_v, w_v, rows_v, out_v,
          sem_iw0, sem_iw1, sem_g0, sem_g1, sem_o0, sem_o1):
        sem_iw = (sem_iw0, sem_iw1)
        sem_g = (sem_g0, sem_g1)
        sem_o = (sem_o0, sem_o1)
        wid = lax.axis_index("s") * NSC + lax.axis_index("c")
        base = wid * PPS

        def start_iw(it, b):
            blk = wid * NIT + it
            pltpu.async_copy(idx_h.at[pl.ds(blk * BLK, BLK)], idx_v.at[b],
                             sem_iw[b])
            pltpu.async_copy(w_h.at[pl.ds(blk * BLK, BLK)], w_v.at[b],
                             sem_iw[b])

        def wait_iw(b):
            pltpu.make_async_copy(idx_h.at[pl.ds(0, BLK)], idx_v.at[b],
                                  sem_iw[b]).wait()
            pltpu.make_async_copy(w_h.at[pl.ds(0, BLK)], w_v.at[b],
                                  sem_iw[b]).wait()

        def start_g(b):
            for j in range(8):
                pltpu.async_copy(
                    table_h.at[idx_v.at[b, pl.ds(j * PB, PB)]],
                    rows_v.at[b, j], sem_g[b])

        def wait_g(b):
            for j in range(8):
                pltpu.make_async_copy(table_h.at[pl.ds(0, PB)],
                                      rows_v.at[b, j], sem_g[b]).wait()

        def start_out(it, b):
            pltpu.async_copy(out_v.at[b],
                             out_h.at[pl.ds(base + it * PB, PB)], sem_o[b])

        def wait_out(b):
            pltpu.make_async_copy(out_h.at[pl.ds(0, PB)], out_v.at[b],
                                  sem_o[b]).wait()

        def compute(b):
            for g in range(PB // 16):
                wvecs = [w_v[b, pl.ds(j * PB + g * 16, 16)] for j in range(8)]
                for pp in range(16):
                    p = g * 16 + pp
                    acc0 = wvecs[0][pp] * rows_v[b, 0, p, 0:16]
                    acc1 = wvecs[0][pp] * rows_v[b, 0, p, 16:32]
                    for j in range(1, 8):
                        wj = wvecs[j][pp]
                        acc0 = acc0 + wj * rows_v[b, j, p, 0:16]
                        acc1 = acc1 + wj * rows_v[b, j, p, 16:32]
                    out_v[b, p, 0:16] = acc0
                    out_v[b, p, 16:32] = acc1

        start_iw(0, 0)
        wait_iw(0)
        start_g(0)
        start_iw(1, 1)

        def step(it, b, last):
            nb = 1 - b
            if not last:
                wait_iw(nb)
                start_g(nb)
            wait_g(b)

            @pl.when(it >= 2)
            def _():
                wait_out(b)

            compute(b)
            start_out(it, b)
            if last:
                return

            @pl.when(it + 2 < NIT)
            def _():
                start_iw(it + 2, b)

        def body(i2, carry):
            step(i2 * 2, 0, False)
            step(i2 * 2 + 1, 1, False)
            return carry

        lax.fori_loop(0, (NIT - 1) // 2, body, 0)
        step(NIT - 1, (NIT - 1) % 2, True)
        wait_out(0)
        wait_out(1)

    return k(table, idx, w)


def _table_body(d_ref, t_ref):
    x = d_ref[...]
    xt = jnp.swapaxes(x, 0, 1)
    vk = xt.shape[0]
    t_ref[...] = jnp.concatenate(
        [xt, jnp.zeros((vk, C - DATA_DIM), jnp.float32)], axis=1)


def _table(d0):
    vk = 8192
    grid = (NVOX // vk,)
    return pl.pallas_call(
        _table_body,
        grid=grid,
        in_specs=[pl.BlockSpec((DATA_DIM, vk), lambda v: (0, v))],
        out_specs=pl.BlockSpec((vk, C), lambda v: (v, 0)),
        out_shape=jax.ShapeDtypeStruct((NVOX, C), jnp.float32),
    )(d0)


def _comp_body(o_ref, d_ref, itp_ref, out_ref):
    o = o_ref[...]
    d = d_ref[...]
    rb = o.shape[0]
    tnear, _ = _ray_bounds(o, d)
    k = lax.broadcasted_iota(jnp.int32, (rb, N_INTRS), 1).astype(jnp.float32)
    ints0 = tnear[:, None] + k * STEP
    ints1 = tnear[:, None] + (k + 1.0) * STEP
    dnorm = jnp.sqrt(jnp.sum(d * d, axis=-1))
    dists = (ints1 - ints0) * dnorm[:, None]

    sigma = jnp.maximum(itp_ref[DATA_DIM - 1], 0.0)
    alpha = 1.0 - jnp.exp(-sigma * dists)
    logt = jnp.log((1.0 - alpha) + 1e-10)
    rr = lax.broadcasted_iota(jnp.int32, (N_INTRS, N_INTRS), 0)
    cc = lax.broadcasted_iota(jnp.int32, (N_INTRS, N_INTRS), 1)
    upper = (rr < cc).astype(jnp.float32)
    cse = jnp.dot(logt, upper, preferred_element_type=jnp.float32)
    abs_light = alpha * jnp.exp(cse)

    x, y, z = d[:, 0], d[:, 1], d[:, 2]
    shm = [jnp.full_like(x, C0), -C1 * y, C1 * z, -C1 * x,
           C2[0] * x * y, C2[1] * y * z,
           C2[2] * (2.0 * z * z - x * x - y * y),
           C2[3] * x * z, C2[4] * (x * x - y * y)]
    acc_sum = jnp.sum(abs_light, axis=-1)
    comps = []
    for c in range(3):
        rgb = shm[0][:, None] * itp_ref[c * SH_DIM]
        for kk in range(1, SH_DIM):
            rgb = rgb + shm[kk][:, None] * itp_ref[c * SH_DIM + kk]
        comps.append(jnp.sum(abs_light * jax.nn.sigmoid(rgb), axis=-1)
                     + (1.0 - acc_sum))
    out_ref[...] = jnp.stack(comps, axis=-1)


def _comp(rays_o, rays_d, interp_t):
    rb = 128
    grid = (BATCH // rb,)
    return pl.pallas_call(
        _comp_body,
        grid=grid,
        in_specs=[pl.BlockSpec((rb, 3), lambda r: (r, 0)),
                  pl.BlockSpec((rb, 3), lambda r: (r, 0)),
                  pl.BlockSpec((DATA_DIM, rb, N_INTRS), lambda r: (0, r, 0))],
        out_specs=pl.BlockSpec((rb, 3), lambda r: (r, 0)),
        out_shape=jax.ShapeDtypeStruct((BATCH, 3), jnp.float32),
    )(rays_o, rays_d, interp_t)


def _iter_major(a):
    return a.reshape(8, NW, NIT, PB).transpose(1, 2, 0, 3).reshape(-1)


def kernel(rays_o, rays_d, data):
    d0 = data[0].reshape(DATA_DIM, NVOX)
    table = _table(d0)
    idx, w = _prep(rays_o, rays_d)
    interp = _sc_interp(table, _iter_major(idx), _iter_major(w))
    interp_t = interp.T[:DATA_DIM].reshape(DATA_DIM, BATCH, N_INTRS)
    return _comp(rays_o, rays_d, interp_t)


# speedup vs baseline: 3.1378x; 3.1378x over previous
import functools

import jax
import jax.numpy as jnp
from jax import lax
from jax.experimental import pallas as pl
from jax.experimental.pallas import tpu as pltpu
from jax.experimental.pallas import tpu_sc as plsc

RES = 128
RADIUS = 1.3
SH_DIM = 9
DATA_DIM = 28
VOXEL = RADIUS * 2 / RES
STEP = VOXEL / 2
N_INTRS = 443
BATCH = 1024
NPTS = BATCH * N_INTRS
C = 32
NVOX = RES * RES * RES

C0 = 0.28209479177387814
C1 = 0.4886025119029199
C2 = (1.0925484305920792, -1.0925484305920792, 0.31539156525252005,
      -1.0925484305920792, 0.5462742152960396)

NSC = 2
NSUB = 16
NW = NSC * NSUB
PPS = NPTS // NW
PB = 32
NIT = PPS // PB


def _ray_bounds(o, d):
    inv = 1.0 / d
    t1 = (-RADIUS - o) * inv
    t2 = (RADIUS - o) * inv
    tnear = jnp.maximum(jnp.max(jnp.minimum(t1, t2), axis=-1), 0.0)
    tfar = jnp.min(jnp.maximum(t1, t2), axis=-1)
    return tnear, tfar


def _prep_body(o_ref, d_ref, idx_ref, w_ref):
    o = o_ref[...]
    d = d_ref[...]
    rb = o.shape[0]
    tnear, tfar = _ray_bounds(o, d)
    k = lax.broadcasted_iota(jnp.int32, (rb, N_INTRS), 1).astype(jnp.float32)
    ints0 = tnear[:, None] + k * STEP
    ints1 = tnear[:, None] + (k + 1.0) * STEP
    tmid = 0.5 * (ints0 + ints1)
    mask = tmid < tfar[:, None]
    i0s, i1s, ws = [], [], []
    for a in range(3):
        pa = o[:, a:a + 1] + d[:, a:a + 1] * tmid
        mask = mask & (jnp.abs(pa) <= RADIUS)
        g = (pa / RADIUS + 1.0) * 0.5 * (RES - 1)
        g0 = jnp.floor(g)
        ws.append(g - g0)
        gi = g0.astype(jnp.int32)
        i0s.append(jnp.clip(gi, 0, RES - 1))
        i1s.append(jnp.clip(gi + 1, 0, RES - 1))
    mf = mask.astype(jnp.float32)
    wx, wy, wz = ws
    for j in range(8):
        jx, jy, jz = j & 1, (j >> 1) & 1, (j >> 2) & 1
        ix = i1s[0] if jx else i0s[0]
        iy = i1s[1] if jy else i0s[1]
        iz = i1s[2] if jz else i0s[2]
        wj = ((wx if jx else 1.0 - wx)
              * (wy if jy else 1.0 - wy)
              * (wz if jz else 1.0 - wz) * mf)
        idx_ref[j] = (iz * RES + iy) * RES + ix
        w_ref[j] = wj


def _prep(rays_o, rays_d):
    rb = 128
    grid = (BATCH // rb,)
    return pl.pallas_call(
        _prep_body,
        grid=grid,
        in_specs=[pl.BlockSpec((rb, 3), lambda r: (r, 0)),
                  pl.BlockSpec((rb, 3), lambda r: (r, 0))],
        out_specs=[pl.BlockSpec((8, rb, N_INTRS), lambda r: (0, r, 0)),
                   pl.BlockSpec((8, rb, N_INTRS), lambda r: (0, r, 0))],
        out_shape=[jax.ShapeDtypeStruct((8, BATCH, N_INTRS), jnp.int32),
                   jax.ShapeDtypeStruct((8, BATCH, N_INTRS), jnp.float32)],
    )(rays_o, rays_d)


BLK = 8 * PB


def _sc_interp(table, idx, w):
    mesh = plsc.VectorSubcoreMesh(core_axis_name="c", subcore_axis_name="s")

    @functools.partial(
        pl.kernel, mesh=mesh,
        compiler_params=pltpu.CompilerParams(use_tc_tiling_on_sc=False),
        out_type=jax.ShapeDtypeStruct((NPTS, C), jnp.float32),
        scratch_types=[
            pltpu.VMEM((2, BLK), jnp.int32),
            pltpu.VMEM((2, BLK), jnp.float32),
            pltpu.VMEM((2, 8, PB, C), jnp.float32),
            pltpu.VMEM((2, PB, C), jnp.float32),
            pltpu.SemaphoreType.DMA,
            pltpu.SemaphoreType.DMA,
            pltpu.SemaphoreType.DMA,
            pltpu.SemaphoreType.DMA,
            pltpu.SemaphoreType.DMA,
            pltpu.SemaphoreType.DMA,
        ],
    )
    def k(table_h, idx_h, w_h, out_h, idx_v, w_v, rows_v, out_v,
          sem_iw0, sem_iw1, sem_g0, sem_g1, sem_o0, sem_o1):
        sem_iw = (sem_iw0, sem_iw1)
        sem_g = (sem_g0, sem_g1)
        sem_o = (sem_o0, sem_o1)
        wid = lax.axis_index("s") * NSC + lax.axis_index("c")
        base = wid * PPS

        def start_iw(it, b):
            blk = wid * NIT + it
            pltpu.async_copy(idx_h.at[pl.ds(blk * BLK, BLK)], idx_v.at[b],
                             sem_iw[b])
            pltpu.async_copy(w_h.at[pl.ds(blk * BLK, BLK)], w_v.at[b],
                             sem_iw[b])

        def wait_iw(b):
            pltpu.make_async_copy(idx_h.at[pl.ds(0, BLK)], idx_v.at[b],
                                  sem_iw[b]).wait()
            pltpu.make_async_copy(w_h.at[pl.ds(0, BLK)], w_v.at[b],
                                  sem_iw[b]).wait()

        def start_g(b):
            for j in range(8):
                pltpu.async_copy(
                    table_h.at[idx_v.at[b, pl.ds(j * PB, PB)]],
                    rows_v.at[b, j], sem_g[b])

        def wait_g(b):
            for j in range(8):
                pltpu.make_async_copy(table_h.at[pl.ds(0, PB)],
                                      rows_v.at[b, j], sem_g[b]).wait()

        def start_out(it, b):
            pltpu.async_copy(out_v.at[b],
                             out_h.at[pl.ds(base + it * PB, PB)], sem_o[b])

        def wait_out(b):
            pltpu.make_async_copy(out_h.at[pl.ds(0, PB)], out_v.at[b],
                                  sem_o[b]).wait()

        def compute(b):
            for g in range(PB // 16):
                wvecs = [w_v[b, pl.ds(j * PB + g * 16, 16)] for j in range(8)]
                for pp in range(16):
                    p = g * 16 + pp
                    acc0 = wvecs[0][pp] * rows_v[b, 0, p, 0:16]
                    acc1 = wvecs[0][pp] * rows_v[b, 0, p, 16:32]
                    for j in range(1, 8):
                        wj = wvecs[j][pp]
                        acc0 = acc0 + wj * rows_v[b, j, p, 0:16]
                        acc1 = acc1 + wj * rows_v[b, j, p, 16:32]
                    out_v[b, p, 0:16] = acc0
                    out_v[b, p, 16:32] = acc1

        start_iw(0, 0)
        wait_iw(0)
        start_g(0)
        start_iw(1, 1)

        def step(it, b, last):
            nb = 1 - b
            if not last:
                wait_iw(nb)
                start_g(nb)
            wait_g(b)

            @pl.when(it >= 2)
            def _():
                wait_out(b)

            compute(b)
            start_out(it, b)
            if last:
                return

            @pl.when(it + 2 < NIT)
            def _():
                start_iw(it + 2, b)

        def body(i2, carry):
            step(i2 * 2, 0, False)
            step(i2 * 2 + 1, 1, False)
            return carry

        lax.fori_loop(0, (NIT - 1) // 2, body, 0)
        step(NIT - 1, (NIT - 1) % 2, True)
        wait_out(0)
        wait_out(1)

    return k(table, idx, w)


VB = 512
VPW = NVOX // NW
NCH = VPW // VB


def _sc_table(data_flat):
    mesh = plsc.VectorSubcoreMesh(core_axis_name="c", subcore_axis_name="s")

    @functools.partial(
        pl.kernel, mesh=mesh,
        compiler_params=pltpu.CompilerParams(use_tc_tiling_on_sc=False,
                                             needs_layout_passes=False),
        out_type=jax.ShapeDtypeStruct((NVOX * C,), jnp.float32),
        scratch_types=[
            pltpu.VMEM((C * VB,), jnp.float32),
            pltpu.VMEM((C * VB,), jnp.float32),
            pltpu.VMEM((VB * C,), jnp.float32),
            pltpu.VMEM((VB * C,), jnp.float32),
            pltpu.SemaphoreType.DMA,
            pltpu.SemaphoreType.DMA,
            pltpu.SemaphoreType.DMA,
            pltpu.SemaphoreType.DMA,
        ],
    )
    def k(d_h, t_h, stage_v0, stage_v1, out_v0, out_v1,
          sem_s0, sem_s1, sem_o0, sem_o1):
        stage = (stage_v0, stage_v1)
        outb = (out_v0, out_v1)
        sem_s = (sem_s0, sem_s1)
        sem_o = (sem_o0, sem_o1)
        wid = lax.axis_index("s") * NSC + lax.axis_index("c")
        v0 = wid * VPW
        zero16 = jnp.zeros((16,), jnp.float32)
        for b in range(2):
            for r in range(DATA_DIM, C):
                for q in range(VB // 16):
                    stage[b][pl.ds(r * VB + q * 16, 16)] = zero16

        def start_in(ck, b):
            off = v0 + ck * VB
            for c in range(DATA_DIM):
                pltpu.async_copy(d_h.at[pl.ds(c * NVOX + off, VB)],
                                 stage[b].at[pl.ds(c * VB, VB)], sem_s[b])

        def wait_in(b):
            for c in range(DATA_DIM):
                pltpu.make_async_copy(d_h.at[pl.ds(0, VB)],
                                      stage[b].at[pl.ds(c * VB, VB)],
                                      sem_s[b]).wait()

        def start_out(ck, b):
            pltpu.async_copy(outb[b],
                             t_h.at[pl.ds((v0 + ck * VB) * C, VB * C)],
                             sem_o[b])

        def wait_out(b):
            pltpu.make_async_copy(t_h.at[pl.ds(0, VB * C)], outb[b],
                                  sem_o[b]).wait()

        base0 = lax.broadcasted_iota(jnp.int32, (16,), 0) * VB
        base1 = base0 + 16 * VB

        def compute(b):
            def cbody(v, carry):
                row0 = plsc.load_gather(stage[b], [base0 + v])
                row1 = plsc.load_gather(stage[b], [base1 + v])
                outb[b][pl.ds(v * C, 16)] = row0
                outb[b][pl.ds(v * C + 16, 16)] = row1
                return carry
            lax.fori_loop(0, VB, cbody, 0, unroll=4)

        start_in(0, 0)
        start_in(1, 1)

        def step(ck, b):
            wait_in(b)

            @pl.when(ck >= 2)
            def _():
                wait_out(b)

            compute(b)
            start_out(ck, b)

            @pl.when(ck + 2 < NCH)
            def _():
                start_in(ck + 2, b)

        def body(c2, carry):
            step(c2 * 2, 0)
            step(c2 * 2 + 1, 1)
            return carry

        lax.fori_loop(0, NCH // 2, body, 0)
        wait_out(0)
        wait_out(1)

    return k(data_flat)


def _comp_body(o_ref, d_ref, itp_ref, out_ref):
    o = o_ref[...]
    d = d_ref[...]
    rb = o.shape[0]
    tnear, _ = _ray_bounds(o, d)
    k = lax.broadcasted_iota(jnp.int32, (rb, N_INTRS), 1).astype(jnp.float32)
    ints0 = tnear[:, None] + k * STEP
    ints1 = tnear[:, None] + (k + 1.0) * STEP
    dnorm = jnp.sqrt(jnp.sum(d * d, axis=-1))
    dists = (ints1 - ints0) * dnorm[:, None]

    sigma = jnp.maximum(itp_ref[DATA_DIM - 1], 0.0)
    alpha = 1.0 - jnp.exp(-sigma * dists)
    logt = jnp.log((1.0 - alpha) + 1e-10)
    rr = lax.broadcasted_iota(jnp.int32, (N_INTRS, N_INTRS), 0)
    cc = lax.broadcasted_iota(jnp.int32, (N_INTRS, N_INTRS), 1)
    upper = (rr < cc).astype(jnp.float32)
    cse = jnp.dot(logt, upper, preferred_element_type=jnp.float32)
    abs_light = alpha * jnp.exp(cse)

    x, y, z = d[:, 0], d[:, 1], d[:, 2]
    shm = [jnp.full_like(x, C0), -C1 * y, C1 * z, -C1 * x,
           C2[0] * x * y, C2[1] * y * z,
           C2[2] * (2.0 * z * z - x * x - y * y),
           C2[3] * x * z, C2[4] * (x * x - y * y)]
    acc_sum = jnp.sum(abs_light, axis=-1)
    comps = []
    for c in range(3):
        rgb = shm[0][:, None] * itp_ref[c * SH_DIM]
        for kk in range(1, SH_DIM):
            rgb = rgb + shm[kk][:, None] * itp_ref[c * SH_DIM + kk]
        comps.append(jnp.sum(abs_light * jax.nn.sigmoid(rgb), axis=-1)
                     + (1.0 - acc_sum))
    out_ref[...] = jnp.stack(comps, axis=-1)


def _comp(rays_o, rays_d, interp_t):
    rb = 128
    grid = (BATCH // rb,)
    return pl.pallas_call(
        _comp_body,
        grid=grid,
        in_specs=[pl.BlockSpec((rb, 3), lambda r: (r, 0)),
                  pl.BlockSpec((rb, 3), lambda r: (r, 0)),
                  pl.BlockSpec((DATA_DIM, rb, N_INTRS), lambda r: (0, r, 0))],
        out_specs=pl.BlockSpec((rb, 3), lambda r: (r, 0)),
        out_shape=jax.ShapeDtypeStruct((BATCH, 3), jnp.float32),
    )(rays_o, rays_d, interp_t)


def _iter_major(a):
    return a.reshape(8, NW, NIT, PB).transpose(1, 2, 0, 3).reshape(-1)


def kernel(rays_o, rays_d, data):
    table = _sc_table(data.reshape(-1)).reshape(NVOX, C)
    idx, w = _prep(rays_o, rays_d)
    interp = _sc_interp(table, _iter_major(idx), _iter_major(w))
    interp_t = interp.T[:DATA_DIM].reshape(DATA_DIM, BATCH, N_INTRS)
    return _comp(rays_o, rays_d, interp_t)


# speedup vs baseline: 3.1551x; 1.0055x over previous
import functools

import jax
import jax.numpy as jnp
from jax import lax
from jax.experimental import pallas as pl
from jax.experimental.pallas import tpu as pltpu
from jax.experimental.pallas import tpu_sc as plsc

RES = 128
RADIUS = 1.3
SH_DIM = 9
DATA_DIM = 28
VOXEL = RADIUS * 2 / RES
STEP = VOXEL / 2
N_INTRS = 443
BATCH = 1024
NPTS = BATCH * N_INTRS
C = 32
NVOX = RES * RES * RES

C0 = 0.28209479177387814
C1 = 0.4886025119029199
C2 = (1.0925484305920792, -1.0925484305920792, 0.31539156525252005,
      -1.0925484305920792, 0.5462742152960396)

NSC = 2
NSUB = 16
NW = NSC * NSUB
PPS = NPTS // NW
PB = 32
NIT = PPS // PB


def _ray_bounds(o, d):
    inv = 1.0 / d
    t1 = (-RADIUS - o) * inv
    t2 = (RADIUS - o) * inv
    tnear = jnp.maximum(jnp.max(jnp.minimum(t1, t2), axis=-1), 0.0)
    tfar = jnp.min(jnp.maximum(t1, t2), axis=-1)
    return tnear, tfar


def _prep_body(o_ref, d_ref, idx_ref, w_ref):
    o = o_ref[...]
    d = d_ref[...]
    rb = o.shape[0]
    tnear, tfar = _ray_bounds(o, d)
    k = lax.broadcasted_iota(jnp.int32, (rb, N_INTRS), 1).astype(jnp.float32)
    ints0 = tnear[:, None] + k * STEP
    ints1 = tnear[:, None] + (k + 1.0) * STEP
    tmid = 0.5 * (ints0 + ints1)
    mask = tmid < tfar[:, None]
    i0s, i1s, ws = [], [], []
    for a in range(3):
        pa = o[:, a:a + 1] + d[:, a:a + 1] * tmid
        mask = mask & (jnp.abs(pa) <= RADIUS)
        g = (pa / RADIUS + 1.0) * 0.5 * (RES - 1)
        g0 = jnp.floor(g)
        ws.append(g - g0)
        gi = g0.astype(jnp.int32)
        i0s.append(jnp.clip(gi, 0, RES - 1))
        i1s.append(jnp.clip(gi + 1, 0, RES - 1))
    mf = mask.astype(jnp.float32)
    wx, wy, wz = ws
    for j in range(8):
        jx, jy, jz = j & 1, (j >> 1) & 1, (j >> 2) & 1
        ix = i1s[0] if jx else i0s[0]
        iy = i1s[1] if jy else i0s[1]
        iz = i1s[2] if jz else i0s[2]
        wj = ((wx if jx else 1.0 - wx)
              * (wy if jy else 1.0 - wy)
              * (wz if jz else 1.0 - wz) * mf)
        idx_ref[j] = (iz * RES + iy) * RES + ix
        w_ref[j] = wj


def _prep(rays_o, rays_d):
    rb = 128
    grid = (BATCH // rb,)
    return pl.pallas_call(
        _prep_body,
        grid=grid,
        in_specs=[pl.BlockSpec((rb, 3), lambda r: (r, 0)),
                  pl.BlockSpec((rb, 3), lambda r: (r, 0))],
        out_specs=[pl.BlockSpec((8, rb, N_INTRS), lambda r: (0, r, 0)),
                   pl.BlockSpec((8, rb, N_INTRS), lambda r: (0, r, 0))],
        out_shape=[jax.ShapeDtypeStruct((8, BATCH, N_INTRS), jnp.int32),
                   jax.ShapeDtypeStruct((8, BATCH, N_INTRS), jnp.float32)],
    )(rays_o, rays_d)


BLK = 8 * PB


def _sc_interp(table, idx, w):
    mesh = plsc.VectorSubcoreMesh(core_axis_name="c", subcore_axis_name="s")

    @functools.partial(
        pl.kernel, mesh=mesh,
        compiler_params=pltpu.CompilerParams(use_tc_tiling_on_sc=False,
                                             needs_layout_passes=False),
        out_type=jax.ShapeDtypeStruct((NPTS, C), jnp.float32),
        scratch_types=[
            pltpu.VMEM((2, BLK), jnp.int32),
            pltpu.VMEM((2, BLK), jnp.float32),
            pltpu.VMEM((2, 8 * PB, C), jnp.bfloat16),
            pltpu.VMEM((2, PB, C), jnp.float32),
            pltpu.SemaphoreType.DMA,
            pltpu.SemaphoreType.DMA,
            pltpu.SemaphoreType.DMA,
            pltpu.SemaphoreType.DMA,
            pltpu.SemaphoreType.DMA,
            pltpu.SemaphoreType.DMA,
        ],
    )
    def k(table_h, idx_h, w_h, out_h, idx_v, w_v, rows_v, out_v,
          sem_iw0, sem_iw1, sem_g0, sem_g1, sem_o0, sem_o1):
        sem_iw = (sem_iw0, sem_iw1)
        sem_g = (sem_g0, sem_g1)
        sem_o = (sem_o0, sem_o1)
        wid = lax.axis_index("s") * NSC + lax.axis_index("c")
        base = wid * PPS

        def start_iw(it, b):
            blk = wid * NIT + it
            pltpu.async_copy(idx_h.at[pl.ds(blk * BLK, BLK)], idx_v.at[b],
                             sem_iw[b])
            pltpu.async_copy(w_h.at[pl.ds(blk * BLK, BLK)], w_v.at[b],
                             sem_iw[b])

        def wait_iw(b):
            pltpu.make_async_copy(idx_h.at[pl.ds(0, BLK)], idx_v.at[b],
                                  sem_iw[b]).wait()
            pltpu.make_async_copy(w_h.at[pl.ds(0, BLK)], w_v.at[b],
                                  sem_iw[b]).wait()

        def start_g(b):
            for h in range(2):
                pltpu.async_copy(
                    table_h.at[idx_v.at[b, pl.ds(h * 128, 128)]],
                    rows_v.at[b, pl.ds(h * 128, 128)], sem_g[b])

        def wait_g(b):
            for h in range(2):
                pltpu.make_async_copy(table_h.at[pl.ds(0, 128)],
                                      rows_v.at[b, pl.ds(h * 128, 128)],
                                      sem_g[b]).wait()

        def start_out(it, b):
            pltpu.async_copy(out_v.at[b],
                             out_h.at[pl.ds(base + it * PB, PB)], sem_o[b])

        def wait_out(b):
            pltpu.make_async_copy(out_h.at[pl.ds(0, PB)], out_v.at[b],
                                  sem_o[b]).wait()

        def compute(b):
            for g in range(PB // 16):
                wvecs = [w_v[b, pl.ds(j * PB + g * 16, 16)] for j in range(8)]
                for pp in range(16):
                    p = g * 16 + pp
                    acc0 = jnp.zeros((16,), jnp.float32)
                    acc1 = jnp.zeros((16,), jnp.float32)
                    for j in range(8):
                        wj = wvecs[j][pp]
                        r0, r1 = plsc.unpack(
                            rows_v[b, j * PB + p, 0:C],
                            format=plsc.PackFormat.INTERLEAVED)
                        acc0 = acc0 + wj * r0
                        acc1 = acc1 + wj * r1
                    out_v[b, p, 0:16] = acc0
                    out_v[b, p, 16:32] = acc1

        start_iw(0, 0)
        wait_iw(0)
        start_g(0)
        start_iw(1, 1)

        def step(it, b, last):
            nb = 1 - b
            if not last:
                wait_iw(nb)
                start_g(nb)
            wait_g(b)

            @pl.when(it >= 2)
            def _():
                wait_out(b)

            compute(b)
            start_out(it, b)
            if last:
                return

            @pl.when(it + 2 < NIT)
            def _():
                start_iw(it + 2, b)

        def body(i2, carry):
            step(i2 * 2, 0, False)
            step(i2 * 2 + 1, 1, False)
            return carry

        lax.fori_loop(0, (NIT - 1) // 2, body, 0)
        step(NIT - 1, (NIT - 1) % 2, True)
        wait_out(0)
        wait_out(1)

    return k(table, idx, w)


VB = 1024
VPW = NVOX // NW
NCH = VPW // VB


def _sc_table(data_flat):
    mesh = plsc.VectorSubcoreMesh(core_axis_name="c", subcore_axis_name="s")

    @functools.partial(
        pl.kernel, mesh=mesh,
        compiler_params=pltpu.CompilerParams(use_tc_tiling_on_sc=False,
                                             needs_layout_passes=False),
        out_type=jax.ShapeDtypeStruct((NVOX * C,), jnp.bfloat16),
        scratch_types=[
            pltpu.VMEM((C * VB,), jnp.float32),
            pltpu.VMEM((C * VB,), jnp.float32),
            pltpu.VMEM((VB * C,), jnp.bfloat16),
            pltpu.VMEM((VB * C,), jnp.bfloat16),
            pltpu.SemaphoreType.DMA,
            pltpu.SemaphoreType.DMA,
            pltpu.SemaphoreType.DMA,
            pltpu.SemaphoreType.DMA,
        ],
    )
    def k(d_h, t_h, stage_v0, stage_v1, out_v0, out_v1,
          sem_s0, sem_s1, sem_o0, sem_o1):
        stage = (stage_v0, stage_v1)
        outb = (out_v0, out_v1)
        sem_s = (sem_s0, sem_s1)
        sem_o = (sem_o0, sem_o1)
        wid = lax.axis_index("s") * NSC + lax.axis_index("c")
        v0 = wid * VPW
        zero16 = jnp.zeros((16,), jnp.float32)
        for b in range(2):
            for r in range(DATA_DIM, C):
                for q in range(VB // 16):
                    stage[b][pl.ds(r * VB + q * 16, 16)] = zero16

        def start_in(ck, b):
            off = v0 + ck * VB
            for c in range(DATA_DIM):
                pltpu.async_copy(d_h.at[pl.ds(c * NVOX + off, VB)],
                                 stage[b].at[pl.ds(c * VB, VB)], sem_s[b])

        def wait_in(b):
            for c in range(DATA_DIM):
                pltpu.make_async_copy(d_h.at[pl.ds(0, VB)],
                                      stage[b].at[pl.ds(c * VB, VB)],
                                      sem_s[b]).wait()

        def start_out(ck, b):
            pltpu.async_copy(outb[b],
                             t_h.at[pl.ds((v0 + ck * VB) * C, VB * C)],
                             sem_o[b])

        def wait_out(b):
            pltpu.make_async_copy(t_h.at[pl.ds(0, VB * C)], outb[b],
                                  sem_o[b]).wait()

        base0 = lax.broadcasted_iota(jnp.int32, (16,), 0) * VB
        base1 = base0 + 16 * VB

        def compute(b):
            def cbody(v, carry):
                row0 = plsc.load_gather(stage[b], [base0 + v])
                row1 = plsc.load_gather(stage[b], [base1 + v])
                outb[b][pl.ds(v * C, C)] = plsc.pack(
                    row0, row1, format=plsc.PackFormat.INTERLEAVED)
                return carry
            lax.fori_loop(0, VB, cbody, 0, unroll=4)

        start_in(0, 0)
        start_in(1, 1)

        def step(ck, b):
            wait_in(b)

            @pl.when(ck >= 2)
            def _():
                wait_out(b)

            compute(b)
            start_out(ck, b)

            @pl.when(ck + 2 < NCH)
            def _():
                start_in(ck + 2, b)

        def body(c2, carry):
            step(c2 * 2, 0)
            step(c2 * 2 + 1, 1)
            return carry

        lax.fori_loop(0, NCH // 2, body, 0)
        wait_out(0)
        wait_out(1)

    return k(data_flat)


def _comp_body(o_ref, d_ref, itp_ref, out_ref):
    o = o_ref[...]
    d = d_ref[...]
    rb = o.shape[0]
    tnear, _ = _ray_bounds(o, d)
    k = lax.broadcasted_iota(jnp.int32, (rb, N_INTRS), 1).astype(jnp.float32)
    ints0 = tnear[:, None] + k * STEP
    ints1 = tnear[:, None] + (k + 1.0) * STEP
    dnorm = jnp.sqrt(jnp.sum(d * d, axis=-1))
    dists = (ints1 - ints0) * dnorm[:, None]

    sigma = jnp.maximum(itp_ref[DATA_DIM - 1], 0.0)
    alpha = 1.0 - jnp.exp(-sigma * dists)
    logt = jnp.log((1.0 - alpha) + 1e-10)
    rr = lax.broadcasted_iota(jnp.int32, (N_INTRS, N_INTRS), 0)
    cc = lax.broadcasted_iota(jnp.int32, (N_INTRS, N_INTRS), 1)
    upper = (rr < cc).astype(jnp.float32)
    cse = jnp.dot(logt, upper, preferred_element_type=jnp.float32)
    abs_light = alpha * jnp.exp(cse)

    x, y, z = d[:, 0], d[:, 1], d[:, 2]
    shm = [jnp.full_like(x, C0), -C1 * y, C1 * z, -C1 * x,
           C2[0] * x * y, C2[1] * y * z,
           C2[2] * (2.0 * z * z - x * x - y * y),
           C2[3] * x * z, C2[4] * (x * x - y * y)]
    acc_sum = jnp.sum(abs_light, axis=-1)
    comps = []
    for c in range(3):
        rgb = shm[0][:, None] * itp_ref[c * SH_DIM]
        for kk in range(1, SH_DIM):
            rgb = rgb + shm[kk][:, None] * itp_ref[c * SH_DIM + kk]
        comps.append(jnp.sum(abs_light * jax.nn.sigmoid(rgb), axis=-1)
                     + (1.0 - acc_sum))
    out_ref[...] = jnp.stack(comps, axis=-1)


def _comp(rays_o, rays_d, interp_t):
    rb = 128
    grid = (BATCH // rb,)
    return pl.pallas_call(
        _comp_body,
        grid=grid,
        in_specs=[pl.BlockSpec((rb, 3), lambda r: (r, 0)),
                  pl.BlockSpec((rb, 3), lambda r: (r, 0)),
                  pl.BlockSpec((DATA_DIM, rb, N_INTRS), lambda r: (0, r, 0))],
        out_specs=pl.BlockSpec((rb, 3), lambda r: (r, 0)),
        out_shape=jax.ShapeDtypeStruct((BATCH, 3), jnp.float32),
    )(rays_o, rays_d, interp_t)


def _iter_major(a):
    return a.reshape(8, NW, NIT, PB).transpose(1, 2, 0, 3).reshape(-1)


def kernel(rays_o, rays_d, data):
    table = _sc_table(data.reshape(-1)).reshape(NVOX, C)
    idx, w = _prep(rays_o, rays_d)
    interp = _sc_interp(table, _iter_major(idx), _iter_major(w))
    interp_t = interp.T[:DATA_DIM].reshape(DATA_DIM, BATCH, N_INTRS)
    return _comp(rays_o, rays_d, interp_t)


# speedup vs baseline: 3.1659x; 1.0034x over previous
import functools

import jax
import jax.numpy as jnp
from jax import lax
from jax.experimental import pallas as pl
from jax.experimental.pallas import tpu as pltpu
from jax.experimental.pallas import tpu_sc as plsc

RES = 128
RADIUS = 1.3
SH_DIM = 9
DATA_DIM = 28
VOXEL = RADIUS * 2 / RES
STEP = VOXEL / 2
N_INTRS = 443
BATCH = 1024
NPTS = BATCH * N_INTRS
C = 32
NVOX = RES * RES * RES

C0 = 0.28209479177387814
C1 = 0.4886025119029199
C2 = (1.0925484305920792, -1.0925484305920792, 0.31539156525252005,
      -1.0925484305920792, 0.5462742152960396)

NSC = 2
NSUB = 16
NW = NSC * NSUB
PPS = NPTS // NW
PB = 32
NIT = PPS // PB


def _ray_bounds(o, d):
    inv = 1.0 / d
    t1 = (-RADIUS - o) * inv
    t2 = (RADIUS - o) * inv
    tnear = jnp.maximum(jnp.max(jnp.minimum(t1, t2), axis=-1), 0.0)
    tfar = jnp.min(jnp.maximum(t1, t2), axis=-1)
    return tnear, tfar


def _prep_body(o_ref, d_ref, idx_ref, w_ref):
    o = o_ref[...]
    d = d_ref[...]
    rb = o.shape[0]
    tnear, tfar = _ray_bounds(o, d)
    k = lax.broadcasted_iota(jnp.int32, (rb, N_INTRS), 1).astype(jnp.float32)
    ints0 = tnear[:, None] + k * STEP
    ints1 = tnear[:, None] + (k + 1.0) * STEP
    tmid = 0.5 * (ints0 + ints1)
    mask = tmid < tfar[:, None]
    i0s, i1s, ws = [], [], []
    for a in range(3):
        pa = o[:, a:a + 1] + d[:, a:a + 1] * tmid
        mask = mask & (jnp.abs(pa) <= RADIUS)
        g = (pa / RADIUS + 1.0) * 0.5 * (RES - 1)
        g0 = jnp.floor(g)
        ws.append(g - g0)
        gi = g0.astype(jnp.int32)
        i0s.append(jnp.clip(gi, 0, RES - 1))
        i1s.append(jnp.clip(gi + 1, 0, RES - 1))
    mf = mask.astype(jnp.float32)
    wx, wy, wz = ws
    for j in range(8):
        jx, jy, jz = j & 1, (j >> 1) & 1, (j >> 2) & 1
        ix = i1s[0] if jx else i0s[0]
        iy = i1s[1] if jy else i0s[1]
        iz = i1s[2] if jz else i0s[2]
        wj = ((wx if jx else 1.0 - wx)
              * (wy if jy else 1.0 - wy)
              * (wz if jz else 1.0 - wz) * mf)
        idx_ref[j] = (iz * RES + iy) * RES + ix
        w_ref[j] = wj


def _prep(rays_o, rays_d):
    rb = 128
    grid = (BATCH // rb,)
    return pl.pallas_call(
        _prep_body,
        grid=grid,
        in_specs=[pl.BlockSpec((rb, 3), lambda r: (r, 0)),
                  pl.BlockSpec((rb, 3), lambda r: (r, 0))],
        out_specs=[pl.BlockSpec((8, rb, N_INTRS), lambda r: (0, r, 0)),
                   pl.BlockSpec((8, rb, N_INTRS), lambda r: (0, r, 0))],
        out_shape=[jax.ShapeDtypeStruct((8, BATCH, N_INTRS), jnp.int32),
                   jax.ShapeDtypeStruct((8, BATCH, N_INTRS), jnp.float32)],
    )(rays_o, rays_d)


BLK = 8 * PB


def _sc_interp(table, idx, w):
    mesh = plsc.VectorSubcoreMesh(core_axis_name="c", subcore_axis_name="s")

    @functools.partial(
        pl.kernel, mesh=mesh,
        compiler_params=pltpu.CompilerParams(use_tc_tiling_on_sc=False,
                                             needs_layout_passes=False),
        out_type=jax.ShapeDtypeStruct((NPTS, C), jnp.float32),
        scratch_types=[
            pltpu.VMEM((2, BLK), jnp.int32),
            pltpu.VMEM((2, BLK), jnp.float32),
            pltpu.VMEM((2, 8 * PB, C), jnp.bfloat16),
            pltpu.VMEM((2, PB, C), jnp.float32),
            pltpu.SemaphoreType.DMA,
            pltpu.SemaphoreType.DMA,
            pltpu.SemaphoreType.DMA,
            pltpu.SemaphoreType.DMA,
            pltpu.SemaphoreType.DMA,
            pltpu.SemaphoreType.DMA,
        ],
    )
    def k(table_h, idx_h, w_h, out_h, idx_v, w_v, rows_v, out_v,
          sem_iw0, sem_iw1, sem_g0, sem_g1, sem_o0, sem_o1):
        sem_iw = (sem_iw0, sem_iw1)
        sem_g = (sem_g0, sem_g1)
        sem_o = (sem_o0, sem_o1)
        wid = lax.axis_index("s") * NSC + lax.axis_index("c")
        base = wid * PPS

        def start_iw(it, b):
            blk = wid * NIT + it
            pltpu.async_copy(idx_h.at[pl.ds(blk * BLK, BLK)], idx_v.at[b],
                             sem_iw[b])
            pltpu.async_copy(w_h.at[pl.ds(blk * BLK, BLK)], w_v.at[b],
                             sem_iw[b])

        def wait_iw(b):
            pltpu.make_async_copy(idx_h.at[pl.ds(0, BLK)], idx_v.at[b],
                                  sem_iw[b]).wait()
            pltpu.make_async_copy(w_h.at[pl.ds(0, BLK)], w_v.at[b],
                                  sem_iw[b]).wait()

        def start_g(b):
            for h in range(2):
                pltpu.async_copy(
                    table_h.at[idx_v.at[b, pl.ds(h * 128, 128)]],
                    rows_v.at[b, pl.ds(h * 128, 128)], sem_g[b])

        def wait_g(b):
            for h in range(2):
                pltpu.make_async_copy(table_h.at[pl.ds(0, 128)],
                                      rows_v.at[b, pl.ds(h * 128, 128)],
                                      sem_g[b]).wait()

        def start_out(it, b):
            pltpu.async_copy(out_v.at[b],
                             out_h.at[pl.ds(base + it * PB, PB)], sem_o[b])

        def wait_out(b):
            pltpu.make_async_copy(out_h.at[pl.ds(0, PB)], out_v.at[b],
                                  sem_o[b]).wait()

        def compute(b):
            for g in range(PB // 16):
                wvecs = [w_v[b, pl.ds(j * PB + g * 16, 16)] for j in range(8)]
                for pp in range(16):
                    p = g * 16 + pp
                    acc0 = jnp.zeros((16,), jnp.float32)
                    acc1 = jnp.zeros((16,), jnp.float32)
                    for j in range(8):
                        wj = wvecs[j][pp]
                        r0, r1 = plsc.unpack(
                            rows_v[b, j * PB + p, 0:C],
                            format=plsc.PackFormat.INTERLEAVED)
                        acc0 = acc0 + wj * r0
                        acc1 = acc1 + wj * r1
                    out_v[b, p, 0:16] = acc0
                    out_v[b, p, 16:32] = acc1

        start_iw(0, 0)
        wait_iw(0)
        start_g(0)
        start_iw(1, 1)

        def step(it, b, last):
            nb = 1 - b
            if not last:
                wait_iw(nb)
                start_g(nb)
            wait_g(b)

            @pl.when(it >= 2)
            def _():
                wait_out(b)

            compute(b)
            start_out(it, b)
            if last:
                return

            @pl.when(it + 2 < NIT)
            def _():
                start_iw(it + 2, b)

        def body(i2, carry):
            step(i2 * 2, 0, False)
            step(i2 * 2 + 1, 1, False)
            return carry

        lax.fori_loop(0, (NIT - 1) // 2, body, 0)
        step(NIT - 1, (NIT - 1) % 2, True)
        wait_out(0)
        wait_out(1)

    return k(table, idx, w)


VB = 1024
VPW = NVOX // NW
NCH = VPW // VB


def _sc_table(data_flat):
    mesh = plsc.VectorSubcoreMesh(core_axis_name="c", subcore_axis_name="s")

    @functools.partial(
        pl.kernel, mesh=mesh,
        compiler_params=pltpu.CompilerParams(use_tc_tiling_on_sc=False,
                                             needs_layout_passes=False),
        out_type=jax.ShapeDtypeStruct((NVOX * C,), jnp.bfloat16),
        scratch_types=[
            pltpu.VMEM((C, VB), jnp.float32),
            pltpu.VMEM((C, VB), jnp.float32),
            pltpu.VMEM((VB * C,), jnp.bfloat16),
            pltpu.VMEM((VB * C,), jnp.bfloat16),
            pltpu.SemaphoreType.DMA,
            pltpu.SemaphoreType.DMA,
            pltpu.SemaphoreType.DMA,
            pltpu.SemaphoreType.DMA,
        ],
    )
    def k(d_h, t_h, stage_v0, stage_v1, out_v0, out_v1,
          sem_s0, sem_s1, sem_o0, sem_o1):
        stage = (stage_v0, stage_v1)
        outb = (out_v0, out_v1)
        sem_s = (sem_s0, sem_s1)
        sem_o = (sem_o0, sem_o1)
        wid = lax.axis_index("s") * NSC + lax.axis_index("c")
        v0 = wid * VPW
        zero16 = jnp.zeros((16,), jnp.float32)
        for b in range(2):
            for r in range(DATA_DIM, C):
                for q in range(VB // 16):
                    stage[b][r, pl.ds(q * 16, 16)] = zero16

        def start_in(ck, b):
            off = v0 + ck * VB
            pltpu.async_copy(d_h.at[:, pl.ds(off, VB)],
                             stage[b].at[pl.ds(0, DATA_DIM)], sem_s[b])

        def wait_in(b):
            pltpu.make_async_copy(d_h.at[:, pl.ds(0, VB)],
                                  stage[b].at[pl.ds(0, DATA_DIM)],
                                  sem_s[b]).wait()

        def start_out(ck, b):
            pltpu.async_copy(outb[b],
                             t_h.at[pl.ds((v0 + ck * VB) * C, VB * C)],
                             sem_o[b])

        def wait_out(b):
            pltpu.make_async_copy(t_h.at[pl.ds(0, VB * C)], outb[b],
                                  sem_o[b]).wait()

        rows0 = lax.broadcasted_iota(jnp.int32, (16,), 0)
        rows1 = rows0 + 16
        zi = jnp.zeros((16,), jnp.int32)

        def compute(b):
            def cbody(v, carry):
                cols = zi + v
                row0 = plsc.load_gather(stage[b], [rows0, cols])
                row1 = plsc.load_gather(stage[b], [rows1, cols])
                outb[b][pl.ds(v * C, C)] = plsc.pack(
                    row0, row1, format=plsc.PackFormat.INTERLEAVED)
                return carry
            lax.fori_loop(0, VB, cbody, 0, unroll=4)

        start_in(0, 0)
        start_in(1, 1)

        def step(ck, b):
            wait_in(b)

            @pl.when(ck >= 2)
            def _():
                wait_out(b)

            compute(b)
            start_out(ck, b)

            @pl.when(ck + 2 < NCH)
            def _():
                start_in(ck + 2, b)

        def body(c2, carry):
            step(c2 * 2, 0)
            step(c2 * 2 + 1, 1)
            return carry

        lax.fori_loop(0, NCH // 2, body, 0)
        wait_out(0)
        wait_out(1)

    return k(data_flat)


def _comp_body(o_ref, d_ref, itp_ref, out_ref):
    o = o_ref[...]
    d = d_ref[...]
    rb = o.shape[0]
    tnear, _ = _ray_bounds(o, d)
    k = lax.broadcasted_iota(jnp.int32, (rb, N_INTRS), 1).astype(jnp.float32)
    ints0 = tnear[:, None] + k * STEP
    ints1 = tnear[:, None] + (k + 1.0) * STEP
    dnorm = jnp.sqrt(jnp.sum(d * d, axis=-1))
    dists = (ints1 - ints0) * dnorm[:, None]

    sigma = jnp.maximum(itp_ref[DATA_DIM - 1], 0.0)
    alpha = 1.0 - jnp.exp(-sigma * dists)
    logt = jnp.log((1.0 - alpha) + 1e-10)
    rr = lax.broadcasted_iota(jnp.int32, (N_INTRS, N_INTRS), 0)
    cc = lax.broadcasted_iota(jnp.int32, (N_INTRS, N_INTRS), 1)
    upper = (rr < cc).astype(jnp.float32)
    cse = jnp.dot(logt, upper, preferred_element_type=jnp.float32)
    abs_light = alpha * jnp.exp(cse)

    x, y, z = d[:, 0], d[:, 1], d[:, 2]
    shm = [jnp.full_like(x, C0), -C1 * y, C1 * z, -C1 * x,
           C2[0] * x * y, C2[1] * y * z,
           C2[2] * (2.0 * z * z - x * x - y * y),
           C2[3] * x * z, C2[4] * (x * x - y * y)]
    acc_sum = jnp.sum(abs_light, axis=-1)
    comps = []
    for c in range(3):
        rgb = shm[0][:, None] * itp_ref[c * SH_DIM]
        for kk in range(1, SH_DIM):
            rgb = rgb + shm[kk][:, None] * itp_ref[c * SH_DIM + kk]
        comps.append(jnp.sum(abs_light * jax.nn.sigmoid(rgb), axis=-1)
                     + (1.0 - acc_sum))
    out_ref[...] = jnp.stack(comps, axis=-1)


def _comp(rays_o, rays_d, interp_t):
    rb = 128
    grid = (BATCH // rb,)
    return pl.pallas_call(
        _comp_body,
        grid=grid,
        in_specs=[pl.BlockSpec((rb, 3), lambda r: (r, 0)),
                  pl.BlockSpec((rb, 3), lambda r: (r, 0)),
                  pl.BlockSpec((DATA_DIM, rb, N_INTRS), lambda r: (0, r, 0))],
        out_specs=pl.BlockSpec((rb, 3), lambda r: (r, 0)),
        out_shape=jax.ShapeDtypeStruct((BATCH, 3), jnp.float32),
    )(rays_o, rays_d, interp_t)


def _iter_major(a):
    return a.reshape(8, NW, NIT, PB).transpose(1, 2, 0, 3).reshape(-1)


def kernel(rays_o, rays_d, data):
    table = _sc_table(data.reshape(DATA_DIM, NVOX)).reshape(NVOX, C)
    idx, w = _prep(rays_o, rays_d)
    interp = _sc_interp(table, _iter_major(idx), _iter_major(w))
    interp_t = interp.T[:DATA_DIM].reshape(DATA_DIM, BATCH, N_INTRS)
    return _comp(rays_o, rays_d, interp_t)


# speedup vs baseline: 4.4443x; 1.4038x over previous
import functools

import jax
import jax.numpy as jnp
from jax import lax
from jax.experimental import pallas as pl
from jax.experimental.pallas import tpu as pltpu
from jax.experimental.pallas import tpu_sc as plsc

RES = 128
RADIUS = 1.3
SH_DIM = 9
DATA_DIM = 28
VOXEL = RADIUS * 2 / RES
STEP = VOXEL / 2
N_INTRS = 443
BATCH = 1024
NPTS = BATCH * N_INTRS
C = 32
NVOX = RES * RES * RES

C0 = 0.28209479177387814
C1 = 0.4886025119029199
C2 = (1.0925484305920792, -1.0925484305920792, 0.31539156525252005,
      -1.0925484305920792, 0.5462742152960396)

NSC = 2
NSUB = 16
NW = NSC * NSUB
PPS = NPTS // NW
PB = 32
NIT = PPS // PB


def _ray_bounds(o, d):
    inv = 1.0 / d
    t1 = (-RADIUS - o) * inv
    t2 = (RADIUS - o) * inv
    tnear = jnp.maximum(jnp.max(jnp.minimum(t1, t2), axis=-1), 0.0)
    tfar = jnp.min(jnp.maximum(t1, t2), axis=-1)
    return tnear, tfar


def _prep_body(o_ref, d_ref, idx_ref, w_ref):
    o = o_ref[...]
    d = d_ref[...]
    rb = o.shape[0]
    tnear, tfar = _ray_bounds(o, d)
    k = lax.broadcasted_iota(jnp.int32, (rb, N_INTRS), 1).astype(jnp.float32)
    ints0 = tnear[:, None] + k * STEP
    ints1 = tnear[:, None] + (k + 1.0) * STEP
    tmid = 0.5 * (ints0 + ints1)
    mask = tmid < tfar[:, None]
    i0s, i1s, ws = [], [], []
    for a in range(3):
        pa = o[:, a:a + 1] + d[:, a:a + 1] * tmid
        mask = mask & (jnp.abs(pa) <= RADIUS)
        g = (pa / RADIUS + 1.0) * 0.5 * (RES - 1)
        g0 = jnp.floor(g)
        ws.append(g - g0)
        gi = g0.astype(jnp.int32)
        i0s.append(jnp.clip(gi, 0, RES - 1))
        i1s.append(jnp.clip(gi + 1, 0, RES - 1))
    mf = mask.astype(jnp.float32)
    wx, wy, wz = ws
    for j in range(8):
        jx, jy, jz = j & 1, (j >> 1) & 1, (j >> 2) & 1
        ix = i1s[0] if jx else i0s[0]
        iy = i1s[1] if jy else i0s[1]
        iz = i1s[2] if jz else i0s[2]
        wj = ((wx if jx else 1.0 - wx)
              * (wy if jy else 1.0 - wy)
              * (wz if jz else 1.0 - wz) * mf)
        idx_ref[j] = (iz * RES + iy) * RES + ix
        w_ref[j] = wj


def _prep(rays_o, rays_d):
    rb = 128
    grid = (BATCH // rb,)
    return pl.pallas_call(
        _prep_body,
        grid=grid,
        in_specs=[pl.BlockSpec((rb, 3), lambda r: (r, 0)),
                  pl.BlockSpec((rb, 3), lambda r: (r, 0))],
        out_specs=[pl.BlockSpec((8, rb, N_INTRS), lambda r: (0, r, 0)),
                   pl.BlockSpec((8, rb, N_INTRS), lambda r: (0, r, 0))],
        out_shape=[jax.ShapeDtypeStruct((8, BATCH, N_INTRS), jnp.int32),
                   jax.ShapeDtypeStruct((8, BATCH, N_INTRS), jnp.float32)],
    )(rays_o, rays_d)


BLK = 8 * PB


def _sc_interp(table, idx, w):
    mesh = plsc.VectorSubcoreMesh(core_axis_name="c", subcore_axis_name="s")

    @functools.partial(
        pl.kernel, mesh=mesh,
        compiler_params=pltpu.CompilerParams(use_tc_tiling_on_sc=False,
                                             needs_layout_passes=False),
        out_type=jax.ShapeDtypeStruct((NPTS, C), jnp.float32),
        scratch_types=[
            pltpu.VMEM((2, BLK), jnp.int32),
            pltpu.VMEM((2, BLK), jnp.float32),
            pltpu.VMEM((2, 8 * PB, C), jnp.bfloat16),
            pltpu.VMEM((2, PB, C), jnp.float32),
            pltpu.SemaphoreType.DMA,
            pltpu.SemaphoreType.DMA,
            pltpu.SemaphoreType.DMA,
            pltpu.SemaphoreType.DMA,
            pltpu.SemaphoreType.DMA,
            pltpu.SemaphoreType.DMA,
        ],
    )
    def k(table_h, idx_h, w_h, out_h, idx_v, w_v, rows_v, out_v,
          sem_iw0, sem_iw1, sem_g0, sem_g1, sem_o0, sem_o1):
        sem_iw = (sem_iw0, sem_iw1)
        sem_g = (sem_g0, sem_g1)
        sem_o = (sem_o0, sem_o1)
        wid = lax.axis_index("s") * NSC + lax.axis_index("c")
        base = wid * PPS

        def start_iw(it, b):
            blk = wid * NIT + it
            pltpu.async_copy(idx_h.at[pl.ds(blk * BLK, BLK)], idx_v.at[b],
                             sem_iw[b])
            pltpu.async_copy(w_h.at[pl.ds(blk * BLK, BLK)], w_v.at[b],
                             sem_iw[b])

        def wait_iw(b):
            pltpu.make_async_copy(idx_h.at[pl.ds(0, BLK)], idx_v.at[b],
                                  sem_iw[b]).wait()
            pltpu.make_async_copy(w_h.at[pl.ds(0, BLK)], w_v.at[b],
                                  sem_iw[b]).wait()

        def start_g(b):
            for h in range(2):
                pltpu.async_copy(
                    table_h.at[idx_v.at[b, pl.ds(h * 128, 128)]],
                    rows_v.at[b, pl.ds(h * 128, 128)], sem_g[b])

        def wait_g(b):
            for h in range(2):
                pltpu.make_async_copy(table_h.at[pl.ds(0, 128)],
                                      rows_v.at[b, pl.ds(h * 128, 128)],
                                      sem_g[b]).wait()

        def start_out(it, b):
            pltpu.async_copy(out_v.at[b],
                             out_h.at[pl.ds(base + it * PB, PB)], sem_o[b])

        def wait_out(b):
            pltpu.make_async_copy(out_h.at[pl.ds(0, PB)], out_v.at[b],
                                  sem_o[b]).wait()

        def compute(b):
            for g in range(PB // 16):
                wvecs = [w_v[b, pl.ds(j * PB + g * 16, 16)] for j in range(8)]
                for pp in range(16):
                    p = g * 16 + pp
                    acc0 = jnp.zeros((16,), jnp.float32)
                    acc1 = jnp.zeros((16,), jnp.float32)
                    for j in range(8):
                        wj = wvecs[j][pp]
                        r0, r1 = plsc.unpack(
                            rows_v[b, j * PB + p, 0:C],
                            format=plsc.PackFormat.INTERLEAVED)
                        acc0 = acc0 + wj * r0
                        acc1 = acc1 + wj * r1
                    out_v[b, p, 0:16] = acc0
                    out_v[b, p, 16:32] = acc1

        start_iw(0, 0)
        wait_iw(0)
        start_g(0)
        start_iw(1, 1)

        def step(it, b, last):
            nb = 1 - b
            if not last:
                wait_iw(nb)
                start_g(nb)
            wait_g(b)

            @pl.when(it >= 2)
            def _():
                wait_out(b)

            compute(b)
            start_out(it, b)
            if last:
                return

            @pl.when(it + 2 < NIT)
            def _():
                start_iw(it + 2, b)

        def body(i2, carry):
            step(i2 * 2, 0, False)
            step(i2 * 2 + 1, 1, False)
            return carry

        lax.fori_loop(0, (NIT - 1) // 2, body, 0)
        step(NIT - 1, (NIT - 1) % 2, True)
        wait_out(0)
        wait_out(1)

    return k(table, idx, w)


VB = 1024
VPW = NVOX // NW
NCH = VPW // VB


def _sc_table(data_flat):
    mesh = plsc.VectorSubcoreMesh(core_axis_name="c", subcore_axis_name="s")

    @functools.partial(
        pl.kernel, mesh=mesh,
        compiler_params=pltpu.CompilerParams(use_tc_tiling_on_sc=False,
                                             needs_layout_passes=False),
        out_type=jax.ShapeDtypeStruct((NVOX * C,), jnp.bfloat16),
        scratch_types=[
            pltpu.VMEM((C, VB + 1), jnp.float32),
            pltpu.VMEM((C, VB + 1), jnp.float32),
            pltpu.VMEM((VB * C,), jnp.bfloat16),
            pltpu.VMEM((VB * C,), jnp.bfloat16),
            pltpu.SemaphoreType.DMA,
            pltpu.SemaphoreType.DMA,
            pltpu.SemaphoreType.DMA,
            pltpu.SemaphoreType.DMA,
        ],
    )
    def k(d_h, t_h, stage_v0, stage_v1, out_v0, out_v1,
          sem_s0, sem_s1, sem_o0, sem_o1):
        stage = (stage_v0, stage_v1)
        outb = (out_v0, out_v1)
        sem_s = (sem_s0, sem_s1)
        sem_o = (sem_o0, sem_o1)
        wid = lax.axis_index("s") * NSC + lax.axis_index("c")
        v0 = wid * VPW
        zero16 = jnp.zeros((16,), jnp.float32)
        for b in range(2):
            for r in range(DATA_DIM, C):
                for q in range(VB // 16):
                    stage[b][r, pl.ds(q * 16, 16)] = zero16

        def start_in(ck, b):
            off = v0 + ck * VB
            pltpu.async_copy(d_h.at[:, pl.ds(off, VB)],
                             stage[b].at[pl.ds(0, DATA_DIM), pl.ds(0, VB)],
                             sem_s[b])

        def wait_in(b):
            pltpu.make_async_copy(d_h.at[:, pl.ds(0, VB)],
                                  stage[b].at[pl.ds(0, DATA_DIM),
                                              pl.ds(0, VB)],
                                  sem_s[b]).wait()

        def start_out(ck, b):
            pltpu.async_copy(outb[b],
                             t_h.at[pl.ds((v0 + ck * VB) * C, VB * C)],
                             sem_o[b])

        def wait_out(b):
            pltpu.make_async_copy(t_h.at[pl.ds(0, VB * C)], outb[b],
                                  sem_o[b]).wait()

        rows0 = lax.broadcasted_iota(jnp.int32, (16,), 0)
        rows1 = rows0 + 16
        zi = jnp.zeros((16,), jnp.int32)

        def compute(b):
            def cbody(v, carry):
                cols = zi + v
                row0 = plsc.load_gather(stage[b], [rows0, cols])
                row1 = plsc.load_gather(stage[b], [rows1, cols])
                outb[b][pl.ds(v * C, C)] = plsc.pack(
                    row0, row1, format=plsc.PackFormat.INTERLEAVED)
                return carry
            lax.fori_loop(0, VB, cbody, 0, unroll=4)

        start_in(0, 0)
        start_in(1, 1)

        def step(ck, b):
            wait_in(b)

            @pl.when(ck >= 2)
            def _():
                wait_out(b)

            compute(b)
            start_out(ck, b)

            @pl.when(ck + 2 < NCH)
            def _():
                start_in(ck + 2, b)

        def body(c2, carry):
            step(c2 * 2, 0)
            step(c2 * 2 + 1, 1)
            return carry

        lax.fori_loop(0, NCH // 2, body, 0)
        wait_out(0)
        wait_out(1)

    return k(data_flat)


def _comp_body(o_ref, d_ref, itp_ref, out_ref):
    o = o_ref[...]
    d = d_ref[...]
    rb = o.shape[0]
    tnear, _ = _ray_bounds(o, d)
    k = lax.broadcasted_iota(jnp.int32, (rb, N_INTRS), 1).astype(jnp.float32)
    ints0 = tnear[:, None] + k * STEP
    ints1 = tnear[:, None] + (k + 1.0) * STEP
    dnorm = jnp.sqrt(jnp.sum(d * d, axis=-1))
    dists = (ints1 - ints0) * dnorm[:, None]

    sigma = jnp.maximum(itp_ref[DATA_DIM - 1], 0.0)
    alpha = 1.0 - jnp.exp(-sigma * dists)
    logt = jnp.log((1.0 - alpha) + 1e-10)
    rr = lax.broadcasted_iota(jnp.int32, (N_INTRS, N_INTRS), 0)
    cc = lax.broadcasted_iota(jnp.int32, (N_INTRS, N_INTRS), 1)
    upper = (rr < cc).astype(jnp.float32)
    cse = jnp.dot(logt, upper, preferred_element_type=jnp.float32)
    abs_light = alpha * jnp.exp(cse)

    x, y, z = d[:, 0], d[:, 1], d[:, 2]
    shm = [jnp.full_like(x, C0), -C1 * y, C1 * z, -C1 * x,
           C2[0] * x * y, C2[1] * y * z,
           C2[2] * (2.0 * z * z - x * x - y * y),
           C2[3] * x * z, C2[4] * (x * x - y * y)]
    acc_sum = jnp.sum(abs_light, axis=-1)
    comps = []
    for c in range(3):
        rgb = shm[0][:, None] * itp_ref[c * SH_DIM]
        for kk in range(1, SH_DIM):
            rgb = rgb + shm[kk][:, None] * itp_ref[c * SH_DIM + kk]
        comps.append(jnp.sum(abs_light * jax.nn.sigmoid(rgb), axis=-1)
                     + (1.0 - acc_sum))
    out_ref[...] = jnp.stack(comps, axis=-1)


def _comp(rays_o, rays_d, interp_t):
    rb = 128
    grid = (BATCH // rb,)
    return pl.pallas_call(
        _comp_body,
        grid=grid,
        in_specs=[pl.BlockSpec((rb, 3), lambda r: (r, 0)),
                  pl.BlockSpec((rb, 3), lambda r: (r, 0)),
                  pl.BlockSpec((DATA_DIM, rb, N_INTRS), lambda r: (0, r, 0))],
        out_specs=pl.BlockSpec((rb, 3), lambda r: (r, 0)),
        out_shape=jax.ShapeDtypeStruct((BATCH, 3), jnp.float32),
    )(rays_o, rays_d, interp_t)


def _iter_major(a):
    return a.reshape(8, NW, NIT, PB).transpose(1, 2, 0, 3).reshape(-1)


def kernel(rays_o, rays_d, data):
    table = _sc_table(data.reshape(DATA_DIM, NVOX)).reshape(NVOX, C)
    idx, w = _prep(rays_o, rays_d)
    interp = _sc_interp(table, _iter_major(idx), _iter_major(w))
    interp_t = interp.T[:DATA_DIM].reshape(DATA_DIM, BATCH, N_INTRS)
    return _comp(rays_o, rays_d, interp_t)
